# baseline (speedup 1.0000x reference)
import functools

import jax
import jax.numpy as jnp
from jax import lax
from jax.experimental import pallas as pl
from jax.experimental.pallas import tpu as pltpu
from jax.experimental.pallas import tpu_sc as plsc

N = 10000
E = 320000
D = 128
NC = 2
NS = 16
NW = NC * NS
EPW = E // NW
NP_ = 10240
RPS = NP_ // NS
KS = 128
SPP = 80
EPP = SPP * KS
NBUF = 2

def _sc_degrees_body(src_hbm, dst_hbm, outs_hbm, outd_hbm, sidx, didx, degs, degd):
    c = lax.axis_index("c")
    s = lax.axis_index("s")
    wid = s * NC + c
    base = wid * EPW
    pltpu.sync_copy(src_hbm.at[pl.ds(base, EPW)], sidx)
    pltpu.sync_copy(dst_hbm.at[pl.ds(base, EPW)], didx)

    zeros = jnp.zeros((16,), jnp.float32)

    def zbody(i, carry):
        degs[pl.ds(i * 16, 16)] = zeros
        degd[pl.ds(i * 16, 16)] = zeros
        return carry

    lax.fori_loop(0, N // 16, zbody, 0)

    ones = jnp.ones((16,), jnp.float32)

    def body(i, carry):
        sv = sidx[pl.ds(i * 16, 16)]
        dv = didx[pl.ds(i * 16, 16)]
        plsc.addupdate_scatter(degs, [sv], ones)
        plsc.addupdate_scatter(degd, [dv], ones)
        return carry

    lax.fori_loop(0, EPW // 16, body, 0)

    pltpu.sync_copy(degs, outs_hbm.at[wid])
    pltpu.sync_copy(degd, outd_hbm.at[wid])


def _sc_spmm_body(h_hbm, src_hbm, dst_hbm, zrows_hbm, out_hbm,
                  sring, dring, rows0, rows1, agg_sh,
                  isem0, isem1, gsem0, gsem1):
    c = lax.axis_index("c")
    s = lax.axis_index("s")
    wid = s * NC + c

    rows = (rows0, rows1)
    isems = (isem0, isem1)
    gsems = (gsem0, gsem1)

    def fetch_idx(step, slot, sync=False):
        if sync:
            pltpu.sync_copy(src_hbm.at[wid, step], sring.at[slot])
            pltpu.sync_copy(dst_hbm.at[wid, step], dring.at[slot])
        else:
            pltpu.async_copy(src_hbm.at[wid, step], sring.at[slot], isems[slot])
            pltpu.async_copy(dst_hbm.at[wid, step], dring.at[slot], isems[slot])

    def wait_idx(step, slot):
        pltpu.make_async_copy(src_hbm.at[wid, step], sring.at[slot], isems[slot]).wait()
        pltpu.make_async_copy(dst_hbm.at[wid, step], dring.at[slot], isems[slot]).wait()

    def issue_gather(slot):
        pltpu.async_copy(h_hbm.at[sring.at[slot]], rows[slot], gsems[slot])

    def wait_gather(slot):
        pltpu.make_async_copy(h_hbm.at[sring.at[slot]], rows[slot], gsems[slot]).wait()

    def scatter(slot):
        pltpu.sync_copy(rows[slot], agg_sh.at[dring.at[slot]], add=True)

    pltpu.sync_copy(zrows_hbm, agg_sh.at[pl.ds(s * RPS, RPS)])
    plsc.subcore_barrier()

    fetch_idx(0, 0, sync=True)
    issue_gather(0)

    def body(j, carry):
        i = j * 2
        for b in range(2):
            b1 = 1 - b
            fetch_idx(i + b + 1, b1, sync=True)
            issue_gather(b1)
            wait_gather(b)
            scatter(b)
        return carry

    lax.fori_loop(0, (SPP - 2) // 2, body, 0)

    fetch_idx(SPP - 1, 1, sync=True)
    issue_gather(1)
    wait_gather(0)
    scatter(0)
    wait_gather(1)
    scatter(1)
    plsc.subcore_barrier()

    pltpu.sync_copy(agg_sh.at[pl.ds(s * RPS, RPS)],
                    out_hbm.at[c, pl.ds(s * RPS, RPS)])


@functools.lru_cache(maxsize=1)
def _build_sc_kernels():
    mesh = plsc.VectorSubcoreMesh(
        core_axis_name="c", subcore_axis_name="s",
        num_cores=NC, num_subcores=NS)
    sc_degrees = pl.kernel(
        _sc_degrees_body,
        out_type=(jax.ShapeDtypeStruct((NW, N), jnp.float32),
                  jax.ShapeDtypeStruct((NW, N), jnp.float32)),
        mesh=mesh,
        scratch_types=[
            pltpu.VMEM((EPW,), jnp.int32),
            pltpu.VMEM((EPW,), jnp.int32),
            pltpu.VMEM((N,), jnp.float32),
            pltpu.VMEM((N,), jnp.float32),
        ],
        compiler_params=pltpu.CompilerParams(needs_layout_passes=False),
    )
    sc_spmm = pl.kernel(
        _sc_spmm_body,
        out_type=jax.ShapeDtypeStruct((NC, NP_, D), jnp.float32),
        mesh=mesh,
        scratch_types=[
            pltpu.VMEM((2, KS), jnp.int32),
            pltpu.VMEM((2, KS), jnp.int32),
            pltpu.VMEM((KS, D), jnp.float32),
            pltpu.VMEM((KS, D), jnp.float32),
            pltpu.VMEM_SHARED((NP_, D), jnp.float32),
            pltpu.SemaphoreType.DMA,
            pltpu.SemaphoreType.DMA,
            pltpu.SemaphoreType.DMA,
            pltpu.SemaphoreType.DMA,
        ],
    )
    return sc_degrees, sc_spmm


def _tc_norms_body(degs_ref, degd_ref, nsd_ref):
    ds_ = jnp.sum(degs_ref[...], axis=0)
    dd_ = jnp.sum(degd_ref[...], axis=0)
    nsd_ref[0, :] = lax.rsqrt(jnp.maximum(ds_, 1.0))
    nsd_ref[1, :] = lax.rsqrt(jnp.maximum(dd_, 1.0))


_tc_norms = pl.pallas_call(
    _tc_norms_body,
    out_shape=jax.ShapeDtypeStruct((2, N), jnp.float32),
)


def _tc_scale_body(x_ref, ns_ref, o_ref):
    o_ref[...] = x_ref[...] * ns_ref[...]


_tc_scale = pl.pallas_call(
    _tc_scale_body,
    out_shape=jax.ShapeDtypeStruct((N, D), jnp.float32),
)


def _tc_mid_body(p_ref, nd_ref, ns_ref, W_ref, b_ref, g_ref, be_ref, o_ref):
    agg = (p_ref[0, :N] + p_ref[1, :N]) * nd_ref[...]
    y = jnp.dot(agg, W_ref[...], preferred_element_type=jnp.float32)
    y = y + b_ref[...][None, :]
    mu = jnp.mean(y, axis=0, keepdims=True)
    var = jnp.mean((y - mu) ** 2, axis=0, keepdims=True)
    yn = (y - mu) * lax.rsqrt(var + 1e-5) * g_ref[...][None, :] + be_ref[...][None, :]
    o_ref[...] = jnp.maximum(yn, 0.0) * ns_ref[...]


_tc_mid = pl.pallas_call(
    _tc_mid_body,
    out_shape=jax.ShapeDtypeStruct((N, D), jnp.float32),
)


def _tc_final_body(p_ref, nd_ref, W_ref, b_ref, o_ref):
    agg = (p_ref[0, :N] + p_ref[1, :N]) * nd_ref[...]
    y = jnp.dot(agg, W_ref[...], preferred_element_type=jnp.float32)
    o_ref[...] = y + b_ref[...][None, :]


_tc_final = pl.pallas_call(
    _tc_final_body,
    out_shape=jax.ShapeDtypeStruct((N, D), jnp.float32),
)


def kernel(x, edge_index, W1, b1, g1, be1, W2, b2, g2, be2, W3, b3):
    src = edge_index[0]
    dst = edge_index[1]
    _sc_degrees, _sc_spmm = _build_sc_kernels()

    degs_p, degd_p = _sc_degrees(src, dst)
    nsd = _tc_norms(degs_p, degd_p)
    ns_col = nsd[0].reshape(N, 1)
    nd_col = nsd[1].reshape(N, 1)

    zrows = jnp.zeros((RPS, D), jnp.float32)
    pad_s = jnp.zeros((NW, EPP - EPW), jnp.int32)
    pad_d = jnp.full((NW, EPP - EPW), N, jnp.int32)
    src3 = jnp.concatenate([src.reshape(NW, EPW), pad_s], axis=1).reshape(NW, SPP, KS)
    dst3 = jnp.concatenate([dst.reshape(NW, EPW), pad_d], axis=1).reshape(NW, SPP, KS)

    h = _tc_scale(x, ns_col)
    p = _sc_spmm(h, src3, dst3, zrows)
    h = _tc_mid(p, nd_col, ns_col, W1, b1, g1, be1)
    p = _sc_spmm(h, src3, dst3, zrows)
    h = _tc_mid(p, nd_col, ns_col, W2, b2, g2, be2)
    p = _sc_spmm(h, src3, dst3, zrows)
    return _tc_final(p, nd_col, W3, b3)


# speedup vs baseline: 2.2243x; 2.2243x over previous
import functools

import jax
import jax.numpy as jnp
from jax import lax
from jax.experimental import pallas as pl
from jax.experimental.pallas import tpu as pltpu
from jax.experimental.pallas import tpu_sc as plsc

N = 10000
E = 320000
D = 128
NC = 2
NS = 16
NW = NC * NS
EPW = E // NW
NP_ = 10240
RPS = NP_ // NS
KE = 80
STEPS = EPW // KE

def _sc_degrees_body(src_hbm, dst_hbm, outs_hbm, outd_hbm, sidx, didx, degs, degd):
    c = lax.axis_index("c")
    s = lax.axis_index("s")
    wid = s * NC + c
    base = wid * EPW
    pltpu.sync_copy(src_hbm.at[pl.ds(base, EPW)], sidx)
    pltpu.sync_copy(dst_hbm.at[pl.ds(base, EPW)], didx)

    zeros = jnp.zeros((16,), jnp.float32)

    def zbody(i, carry):
        degs[pl.ds(i * 16, 16)] = zeros
        degd[pl.ds(i * 16, 16)] = zeros
        return carry

    lax.fori_loop(0, N // 16, zbody, 0)

    ones = jnp.ones((16,), jnp.float32)

    def body(i, carry):
        sv = sidx[pl.ds(i * 16, 16)]
        dv = didx[pl.ds(i * 16, 16)]
        plsc.addupdate_scatter(degs, [sv], ones)
        plsc.addupdate_scatter(degd, [dv], ones)
        return carry

    lax.fori_loop(0, EPW // 16, body, 0)

    pltpu.sync_copy(degs, outs_hbm.at[wid])
    pltpu.sync_copy(degd, outd_hbm.at[wid])


def _sc_spmm_body(h_hbm, src_hbm, dst_hbm, zrows_hbm, out_hbm,
                  sidx0, didx0, sidx1, didx1, rows0, rows1, agg_sh,
                  gsem0, gsem1):
    c = lax.axis_index("c")
    s = lax.axis_index("s")
    wid = s * NC + c
    base = wid * EPW

    sidx = (sidx0, sidx1)
    didx = (didx0, didx1)
    rows = (rows0, rows1)
    gsems = (gsem0, gsem1)

    def fetch_idx(i, b):
        pltpu.sync_copy(src_hbm.at[pl.ds(base + i * KE, KE)], sidx[b])
        pltpu.sync_copy(dst_hbm.at[pl.ds(base + i * KE, KE)], didx[b])

    def issue_gather(b):
        pltpu.async_copy(h_hbm.at[sidx[b]], rows[b], gsems[b])

    def wait_gather(b):
        pltpu.make_async_copy(h_hbm.at[sidx[b]], rows[b], gsems[b]).wait()

    def scatter(b):
        pltpu.sync_copy(rows[b], agg_sh.at[didx[b]], add=True)

    pltpu.sync_copy(zrows_hbm, agg_sh.at[pl.ds(s * RPS, RPS)])
    plsc.subcore_barrier()

    fetch_idx(0, 0)
    issue_gather(0)

    def body(j, carry):
        i = j * 2
        for b in range(2):
            b1 = 1 - b
            fetch_idx(i + b + 1, b1)
            issue_gather(b1)
            wait_gather(b)
            scatter(b)
        return carry

    lax.fori_loop(0, (STEPS - 1) // 2, body, 0)

    wait_gather(0)
    scatter(0)
    plsc.subcore_barrier()

    pltpu.sync_copy(agg_sh.at[pl.ds(s * RPS, RPS)],
                    out_hbm.at[c, pl.ds(s * RPS, RPS)])


@functools.lru_cache(maxsize=1)
def _build_sc_kernels():
    mesh = plsc.VectorSubcoreMesh(
        core_axis_name="c", subcore_axis_name="s",
        num_cores=NC, num_subcores=NS)
    sc_degrees = pl.kernel(
        _sc_degrees_body,
        out_type=(jax.ShapeDtypeStruct((NW, N), jnp.float32),
                  jax.ShapeDtypeStruct((NW, N), jnp.float32)),
        mesh=mesh,
        scratch_types=[
            pltpu.VMEM((EPW,), jnp.int32),
            pltpu.VMEM((EPW,), jnp.int32),
            pltpu.VMEM((N,), jnp.float32),
            pltpu.VMEM((N,), jnp.float32),
        ],
        compiler_params=pltpu.CompilerParams(needs_layout_passes=False),
    )
    sc_spmm = pl.kernel(
        _sc_spmm_body,
        out_type=jax.ShapeDtypeStruct((NC, NP_, D), jnp.float32),
        mesh=mesh,
        scratch_types=[
            pltpu.VMEM((KE,), jnp.int32),
            pltpu.VMEM((KE,), jnp.int32),
            pltpu.VMEM((KE,), jnp.int32),
            pltpu.VMEM((KE,), jnp.int32),
            pltpu.VMEM((KE, D), jnp.float32),
            pltpu.VMEM((KE, D), jnp.float32),
            pltpu.VMEM_SHARED((NP_, D), jnp.float32),
            pltpu.SemaphoreType.DMA,
            pltpu.SemaphoreType.DMA,
        ],
    )
    return sc_degrees, sc_spmm


def _tc_norms_body(degs_ref, degd_ref, nsd_ref):
    ds_ = jnp.sum(degs_ref[...], axis=0)
    dd_ = jnp.sum(degd_ref[...], axis=0)
    nsd_ref[0, :] = lax.rsqrt(jnp.maximum(ds_, 1.0))
    nsd_ref[1, :] = lax.rsqrt(jnp.maximum(dd_, 1.0))


_tc_norms = pl.pallas_call(
    _tc_norms_body,
    out_shape=jax.ShapeDtypeStruct((2, N), jnp.float32),
)


def _tc_scale_body(x_ref, ns_ref, o_ref):
    o_ref[...] = x_ref[...] * ns_ref[...]


_tc_scale = pl.pallas_call(
    _tc_scale_body,
    out_shape=jax.ShapeDtypeStruct((N, D), jnp.float32),
)


def _tc_mid_body(p_ref, nd_ref, ns_ref, W_ref, b_ref, g_ref, be_ref, o_ref):
    agg = (p_ref[0, :N] + p_ref[1, :N]) * nd_ref[...]
    y = jnp.dot(agg, W_ref[...], preferred_element_type=jnp.float32)
    y = y + b_ref[...][None, :]
    mu = jnp.mean(y, axis=0, keepdims=True)
    var = jnp.mean((y - mu) ** 2, axis=0, keepdims=True)
    yn = (y - mu) * lax.rsqrt(var + 1e-5) * g_ref[...][None, :] + be_ref[...][None, :]
    o_ref[...] = jnp.maximum(yn, 0.0) * ns_ref[...]


_tc_mid = pl.pallas_call(
    _tc_mid_body,
    out_shape=jax.ShapeDtypeStruct((N, D), jnp.float32),
)


def _tc_final_body(p_ref, nd_ref, W_ref, b_ref, o_ref):
    agg = (p_ref[0, :N] + p_ref[1, :N]) * nd_ref[...]
    y = jnp.dot(agg, W_ref[...], preferred_element_type=jnp.float32)
    o_ref[...] = y + b_ref[...][None, :]


_tc_final = pl.pallas_call(
    _tc_final_body,
    out_shape=jax.ShapeDtypeStruct((N, D), jnp.float32),
)


def kernel(x, edge_index, W1, b1, g1, be1, W2, b2, g2, be2, W3, b3):
    src = edge_index[0]
    dst = edge_index[1]
    _sc_degrees, _sc_spmm = _build_sc_kernels()

    degs_p, degd_p = _sc_degrees(src, dst)
    nsd = _tc_norms(degs_p, degd_p)
    ns_col = nsd[0].reshape(N, 1)
    nd_col = nsd[1].reshape(N, 1)

    zrows = jnp.zeros((RPS, D), jnp.float32)

    h = _tc_scale(x, ns_col)
    p = _sc_spmm(h, src, dst, zrows)
    h = _tc_mid(p, nd_col, ns_col, W1, b1, g1, be1)
    p = _sc_spmm(h, src, dst, zrows)
    h = _tc_mid(p, nd_col, ns_col, W2, b2, g2, be2)
    p = _sc_spmm(h, src, dst, zrows)
    return _tc_final(p, nd_col, W3, b3)


# speedup vs baseline: 2.5960x; 1.1671x over previous
import functools

import jax
import jax.numpy as jnp
from jax import lax
from jax.experimental import pallas as pl
from jax.experimental.pallas import tpu as pltpu
from jax.experimental.pallas import tpu_sc as plsc

N = 10000
E = 320000
D = 128
NC = 2
NS = 16
NW = NC * NS
EPW = E // NW
NP_ = 10240
RPS = NP_ // NS
KE = 80
STEPS = EPW // KE

def _sc_degrees_body(src_hbm, dst_hbm, outs_hbm, outd_hbm, sidx, didx, degs, degd):
    c = lax.axis_index("c")
    s = lax.axis_index("s")
    wid = s * NC + c
    base = wid * EPW
    pltpu.sync_copy(src_hbm.at[pl.ds(base, EPW)], sidx)
    pltpu.sync_copy(dst_hbm.at[pl.ds(base, EPW)], didx)

    zeros = jnp.zeros((16,), jnp.float32)

    def zbody(i, carry):
        degs[pl.ds(i * 16, 16)] = zeros
        degd[pl.ds(i * 16, 16)] = zeros
        return carry

    lax.fori_loop(0, N // 16, zbody, 0)

    ones = jnp.ones((16,), jnp.float32)

    def body(i, carry):
        sv = sidx[pl.ds(i * 16, 16)]
        dv = didx[pl.ds(i * 16, 16)]
        plsc.addupdate_scatter(degs, [sv], ones)
        plsc.addupdate_scatter(degd, [dv], ones)
        return carry

    lax.fori_loop(0, EPW // 16, body, 0)

    pltpu.sync_copy(degs, outs_hbm.at[wid])
    pltpu.sync_copy(degd, outd_hbm.at[wid])


def _sc_spmm_body(h_hbm, src_hbm, dst_hbm, zrows_hbm, out_hbm,
                  sidx0, didx0, sidx1, didx1, sidx2, didx2,
                  rows0, rows1, rows2, agg_sh,
                  gsem0, gsem1, gsem2, ssem0, ssem1, ssem2):
    c = lax.axis_index("c")
    s = lax.axis_index("s")
    wid = s * NC + c
    base = wid * EPW

    sidx = (sidx0, sidx1, sidx2)
    didx = (didx0, didx1, didx2)
    rows = (rows0, rows1, rows2)
    gsems = (gsem0, gsem1, gsem2)
    ssems = (ssem0, ssem1, ssem2)

    def fetch_idx(i, b):
        pltpu.sync_copy(src_hbm.at[pl.ds(base + i * KE, KE)], sidx[b])
        pltpu.sync_copy(dst_hbm.at[pl.ds(base + i * KE, KE)], didx[b])

    def issue_gather(b):
        pltpu.async_copy(h_hbm.at[sidx[b]], rows[b], gsems[b])

    def wait_gather(b):
        pltpu.make_async_copy(h_hbm.at[sidx[b]], rows[b], gsems[b]).wait()

    def issue_scatter(b):
        pltpu.async_copy(rows[b], agg_sh.at[didx[b]], ssems[b], add=True)

    def wait_scatter(b):
        pltpu.make_async_copy(rows[b], agg_sh.at[didx[b]], ssems[b]).wait()

    pltpu.sync_copy(zrows_hbm, agg_sh.at[pl.ds(s * RPS, RPS)])
    plsc.subcore_barrier()

    fetch_idx(0, 0)
    issue_gather(0)
    fetch_idx(1, 1)
    issue_gather(1)
    wait_gather(0)
    issue_scatter(0)
    fetch_idx(2, 2)
    issue_gather(2)
    wait_gather(1)
    issue_scatter(1)
    wait_scatter(0)
    fetch_idx(3, 0)
    issue_gather(0)

    def body(j, carry):
        i = j * 3
        for k, b in ((2, 2), (3, 0), (4, 1)):
            step = i + k
            wait_gather(b)
            issue_scatter(b)
            wait_scatter((b + 2) % 3)
            fetch_idx(step + 2, (b + 2) % 3)
            issue_gather((b + 2) % 3)
        return carry

    lax.fori_loop(0, (STEPS - 5) // 3, body, 0)

    wait_gather(2)
    issue_scatter(2)
    wait_scatter(1)
    fetch_idx(STEPS - 1, 1)
    issue_gather(1)
    wait_gather(0)
    issue_scatter(0)
    wait_scatter(2)
    wait_gather(1)
    issue_scatter(1)
    wait_scatter(0)
    wait_scatter(1)
    plsc.subcore_barrier()

    pltpu.sync_copy(agg_sh.at[pl.ds(s * RPS, RPS)],
                    out_hbm.at[c, pl.ds(s * RPS, RPS)])


@functools.lru_cache(maxsize=1)
def _build_sc_kernels():
    mesh = plsc.VectorSubcoreMesh(
        core_axis_name="c", subcore_axis_name="s",
        num_cores=NC, num_subcores=NS)
    sc_degrees = pl.kernel(
        _sc_degrees_body,
        out_type=(jax.ShapeDtypeStruct((NW, N), jnp.float32),
                  jax.ShapeDtypeStruct((NW, N), jnp.float32)),
        mesh=mesh,
        scratch_types=[
            pltpu.VMEM((EPW,), jnp.int32),
            pltpu.VMEM((EPW,), jnp.int32),
            pltpu.VMEM((N,), jnp.float32),
            pltpu.VMEM((N,), jnp.float32),
        ],
        compiler_params=pltpu.CompilerParams(needs_layout_passes=False),
    )
    sc_spmm = pl.kernel(
        _sc_spmm_body,
        out_type=jax.ShapeDtypeStruct((NC, NP_, D), jnp.float32),
        mesh=mesh,
        scratch_types=(
            [pltpu.VMEM((KE,), jnp.int32)] * 6
            + [pltpu.VMEM((KE, D), jnp.float32)] * 3
            + [pltpu.VMEM_SHARED((NP_, D), jnp.float32)]
            + [pltpu.SemaphoreType.DMA] * 6
        ),
    )
    return sc_degrees, sc_spmm


def _tc_norms_body(degs_ref, degd_ref, nsd_ref):
    ds_ = jnp.sum(degs_ref[...], axis=0)
    dd_ = jnp.sum(degd_ref[...], axis=0)
    nsd_ref[0, :] = lax.rsqrt(jnp.maximum(ds_, 1.0))
    nsd_ref[1, :] = lax.rsqrt(jnp.maximum(dd_, 1.0))


_tc_norms = pl.pallas_call(
    _tc_norms_body,
    out_shape=jax.ShapeDtypeStruct((2, N), jnp.float32),
)


def _tc_scale_body(x_ref, ns_ref, o_ref):
    o_ref[...] = x_ref[...] * ns_ref[...]


_tc_scale = pl.pallas_call(
    _tc_scale_body,
    out_shape=jax.ShapeDtypeStruct((N, D), jnp.float32),
)


def _tc_mid_body(p_ref, nd_ref, ns_ref, W_ref, b_ref, g_ref, be_ref, o_ref):
    agg = (p_ref[0, :N] + p_ref[1, :N]) * nd_ref[...]
    y = jnp.dot(agg, W_ref[...], preferred_element_type=jnp.float32)
    y = y + b_ref[...][None, :]
    mu = jnp.mean(y, axis=0, keepdims=True)
    var = jnp.mean((y - mu) ** 2, axis=0, keepdims=True)
    yn = (y - mu) * lax.rsqrt(var + 1e-5) * g_ref[...][None, :] + be_ref[...][None, :]
    o_ref[...] = jnp.maximum(yn, 0.0) * ns_ref[...]


_tc_mid = pl.pallas_call(
    _tc_mid_body,
    out_shape=jax.ShapeDtypeStruct((N, D), jnp.float32),
)


def _tc_final_body(p_ref, nd_ref, W_ref, b_ref, o_ref):
    agg = (p_ref[0, :N] + p_ref[1, :N]) * nd_ref[...]
    y = jnp.dot(agg, W_ref[...], preferred_element_type=jnp.float32)
    o_ref[...] = y + b_ref[...][None, :]


_tc_final = pl.pallas_call(
    _tc_final_body,
    out_shape=jax.ShapeDtypeStruct((N, D), jnp.float32),
)


def kernel(x, edge_index, W1, b1, g1, be1, W2, b2, g2, be2, W3, b3):
    src = edge_index[0]
    dst = edge_index[1]
    _sc_degrees, _sc_spmm = _build_sc_kernels()

    degs_p, degd_p = _sc_degrees(src, dst)
    nsd = _tc_norms(degs_p, degd_p)
    ns_col = nsd[0].reshape(N, 1)
    nd_col = nsd[1].reshape(N, 1)

    zrows = jnp.zeros((RPS, D), jnp.float32)

    h = _tc_scale(x, ns_col)
    p = _sc_spmm(h, src, dst, zrows)
    h = _tc_mid(p, nd_col, ns_col, W1, b1, g1, be1)
    p = _sc_spmm(h, src, dst, zrows)
    h = _tc_mid(p, nd_col, ns_col, W2, b2, g2, be2)
    p = _sc_spmm(h, src, dst, zrows)
    return _tc_final(p, nd_col, W3, b3)
